# baseline (speedup 1.0000x reference)
import functools

import jax
import jax.numpy as jnp
from jax import lax
from jax.experimental import pallas as pl
from jax.experimental.pallas import tpu as pltpu
from jax.experimental.pallas import tpu_sc as plsc

_VOCAB = 100000
_EMB = 64
_BATCH = 1024
_HIST = 50
_KEEP_P = 0.7

_NW = 32
_BPW = _BATCH // _NW
_IDX_PER_W = _BPW * _HIST
_CHUNK = 128
_NCHUNK = (_IDX_PER_W + _CHUNK - 1) // _CHUNK
_NCPAD = 16
_NROWS = _NCHUNK * _CHUNK
_STAGES = ((0, 7), (7, 13))
_SBUF = 7 * _CHUNK


def _bcast_idx(idx_v, f):
    row = lax.shift_right_logical(f, 7)
    colg = lax.bitwise_and(lax.shift_right_logical(f, 4), 7) * 16
    iv = idx_v[row, pl.ds(colg, 16)]
    lane = jnp.broadcast_to(lax.bitwise_and(f, 15), (16,))
    return iv.at[lane].get(mode="promise_in_bounds")


def _pool_body(table_hbm, idx_hbm, out_hbm, idx_v, lines_v, rows_v, acc_v, sem):
    wid = lax.axis_index("s") * 2 + lax.axis_index("c")

    pltpu.sync_copy(idx_hbm.at[wid], idx_v)

    for j in range(_NCHUNK):
        for g in range(_CHUNK // 16):
            iv = idx_v[j, pl.ds(g * 16, 16)]
            lines_v[j, pl.ds(g * 16, 16)] = lax.shift_left(
                lax.shift_right_logical(iv, 12), 11
            ) + lax.bitwise_and(iv, 2047)

    for c0, c1 in _STAGES:
        f_lo = c0 * _CHUNK
        f_hi = min(c1 * _CHUNK, _IDX_PER_W)
        copies = []
        for j in range(c0, c1):
            copies.append(
                pltpu.async_copy(
                    table_hbm.at[lines_v.at[j]],
                    rows_v.at[pl.ds((j - c0) * _CHUNK, _CHUNK)],
                    sem,
                )
            )
        for c in copies:
            c.wait()

        for b in range(f_lo // _HIST, (f_hi - 1) // _HIST + 1):
            h0 = max(0, f_lo - b * _HIST)
            h1 = min(_HIST, f_hi - b * _HIST)

            def _acc1(f, acc, f_lo=f_lo):
                ivb = _bcast_idx(idx_v, f)
                hi = lax.bitwise_and(lax.shift_right_logical(ivb, 11), 1)
                hif = lax.convert_element_type(hi, jnp.float32)
                wf = lax.convert_element_type(
                    jnp.minimum(ivb, 1), jnp.float32
                )
                wb = wf * hif
                wa = wf - wb
                out = []
                for c in range(4):
                    va = rows_v[f - f_lo, pl.ds(c * 16, 16)]
                    vb = rows_v[f - f_lo, pl.ds(64 + c * 16, 16)]
                    out.append(acc[c] + va * wa + vb * wb)
                return tuple(out)

            if h0 > 0:
                init = tuple(acc_v[b, pl.ds(c * 16, 16)] for c in range(4))
            else:
                init = tuple(jnp.zeros((16,), jnp.float32) for _ in range(4))

            n = h1 - h0
            base = b * _HIST + h0
            nu, rem = divmod(n, 5)

            def _acc5(i, acc, base=base, _acc1=_acc1):
                f = base + i * 5
                for k in range(5):
                    acc = _acc1(f + k, acc)
                return acc

            acc = lax.fori_loop(0, nu, _acc5, init) if nu else init
            for k in range(n - rem, n):
                acc = _acc1(base + k, acc)
            for c in range(4):
                acc_v[b, pl.ds(c * 16, 16)] = acc[c]

    pltpu.sync_copy(acc_v, out_hbm.at[pl.ds(wid * _BPW, _BPW)])


def _pool(table_lines, idx3):
    mesh = plsc.VectorSubcoreMesh(core_axis_name="c", subcore_axis_name="s")
    return pl.kernel(
        _pool_body,
        mesh=mesh,
        out_type=jax.ShapeDtypeStruct((_BATCH, _EMB), jnp.float32),
        scratch_types=[
            pltpu.VMEM((_NCPAD, _CHUNK), jnp.int32),
            pltpu.VMEM((_NCPAD, _CHUNK), jnp.int32),
            pltpu.VMEM((_SBUF, 2 * _EMB), jnp.float32),
            pltpu.VMEM((_BPW, _EMB), jnp.float32),
            pltpu.SemaphoreType.DMA,
        ],
    )(table_lines, idx3)


_RB = 2048
_NR = 25
_NLINES = _NR * _RB


def _relayout_body(a_ref, b_ref, o_ref):
    o_ref[:, 0:_EMB] = jnp.transpose(a_ref[...])
    o_ref[:, _EMB : 2 * _EMB] = jnp.transpose(b_ref[...])


def _to_lines(emb_table_t):
    return pl.pallas_call(
        _relayout_body,
        grid=(_NR,),
        in_specs=[
            pl.BlockSpec((_EMB, _RB), lambda i: (0, 2 * i)),
            pl.BlockSpec((_EMB, _RB), lambda i: (0, jnp.minimum(2 * i + 1, 48))),
        ],
        out_specs=pl.BlockSpec((_RB, 2 * _EMB), lambda i: (i, 0)),
        out_shape=jax.ShapeDtypeStruct((_NLINES, 2 * _EMB), jnp.float32),
    )(emb_table_t, emb_table_t)


_VT = 2048
_NV = (_VOCAB + _VT - 1) // _VT


def _mm_body(x_ref, m_ref, w_ref, b_ref, o_ref):
    xd = x_ref[...] * m_ref[...]
    acc = lax.dot_general(
        w_ref[...], xd, (((0,), (1,)), ((), ())), preferred_element_type=jnp.float32
    )
    j = pl.program_id(0)
    bias = b_ref[0, pl.ds(j * _VT, _VT)]
    o_ref[...] = acc + bias[:, None]


def _project(pooled, mscale, fc_Wt, bias_padded):
    out_t = pl.pallas_call(
        _mm_body,
        grid=(_NV,),
        in_specs=[
            pl.BlockSpec((_BATCH, _EMB), lambda j: (0, 0)),
            pl.BlockSpec((_BATCH, _EMB), lambda j: (0, 0)),
            pl.BlockSpec((_EMB, _VT), lambda j: (0, j)),
            pl.BlockSpec((1, _NV * _VT), lambda j: (0, 0)),
        ],
        out_specs=pl.BlockSpec((_VT, _BATCH), lambda j: (j, 0)),
        out_shape=jax.ShapeDtypeStruct((_VOCAB, _BATCH), jnp.float32),
    )(pooled, mscale, fc_Wt, bias_padded)
    return out_t.T


def kernel(x_in, emb_table, fc_W, fc_b):
    x = x_in.astype(jnp.int32)
    npad = _NROWS - _IDX_PER_W
    pad = (
        jnp.arange(_NW, dtype=jnp.int32)[:, None] * npad
        + jnp.arange(npad, dtype=jnp.int32)[None, :]
    )
    idx3 = jnp.concatenate([x.reshape(_NW, _IDX_PER_W), pad], axis=1).reshape(
        _NW, _NCHUNK, _CHUNK
    )
    idx3 = jnp.concatenate(
        [idx3, jnp.zeros((_NW, _NCPAD - _NCHUNK, _CHUNK), jnp.int32)], axis=1
    )
    pooled = _pool(_to_lines(emb_table.T), idx3)

    keep = jax.random.bernoulli(jax.random.key(42), _KEEP_P, (_BATCH, _EMB))
    mscale = jnp.where(keep, jnp.float32(1.0 / _KEEP_P), jnp.float32(0.0))

    bias_padded = jnp.pad(fc_b, (0, _NV * _VT - _VOCAB)).reshape(1, -1)
    return _project(pooled, mscale, fc_W.T, bias_padded)


# speedup vs baseline: 1.0064x; 1.0064x over previous
import functools

import jax
import jax.numpy as jnp
from jax import lax
from jax.experimental import pallas as pl
from jax.experimental.pallas import tpu as pltpu
from jax.experimental.pallas import tpu_sc as plsc

_VOCAB = 100000
_EMB = 64
_BATCH = 1024
_HIST = 50
_KEEP_P = 0.7

_NW = 32
_BPW = _BATCH // _NW
_IDX_PER_W = _BPW * _HIST
_CHUNK = 128
_NCHUNK = (_IDX_PER_W + _CHUNK - 1) // _CHUNK
_NCPAD = 16
_NROWS = _NCHUNK * _CHUNK
_STAGES = ((0, 7), (7, 13))
_SBUF = 7 * _CHUNK


def _bcast_idx(idx_v, f):
    row = lax.shift_right_logical(f, 7)
    colg = lax.bitwise_and(lax.shift_right_logical(f, 4), 7) * 16
    iv = idx_v[row, pl.ds(colg, 16)]
    lane = jnp.broadcast_to(lax.bitwise_and(f, 15), (16,))
    return iv.at[lane].get(mode="promise_in_bounds")


def _pool_body(table_hbm, idx_hbm, out_hbm, idx_v, lines_v, rows_v, acc_v, sem):
    wid = lax.axis_index("s") * 2 + lax.axis_index("c")

    pltpu.sync_copy(idx_hbm.at[wid], idx_v)

    for j in range(_NCHUNK):
        for g in range(_CHUNK // 16):
            iv = idx_v[j, pl.ds(g * 16, 16)]
            lines_v[j, pl.ds(g * 16, 16)] = lax.shift_left(
                lax.shift_right_logical(iv, 12), 11
            ) + lax.bitwise_and(iv, 2047)

    for c0, c1 in _STAGES:
        f_lo = c0 * _CHUNK
        f_hi = min(c1 * _CHUNK, _IDX_PER_W)
        copies = []
        for j in range(c0, c1):
            copies.append(
                pltpu.async_copy(
                    table_hbm.at[lines_v.at[j]],
                    rows_v.at[pl.ds((j - c0) * _CHUNK, _CHUNK)],
                    sem,
                )
            )
        for c in copies:
            c.wait()

        for b in range(f_lo // _HIST, (f_hi - 1) // _HIST + 1):
            h0 = max(0, f_lo - b * _HIST)
            h1 = min(_HIST, f_hi - b * _HIST)

            def _acc1(f, acc, f_lo=f_lo):
                ivb = _bcast_idx(idx_v, f)
                hi = lax.bitwise_and(lax.shift_right_logical(ivb, 11), 1)
                hif = lax.convert_element_type(hi, jnp.float32)
                wf = lax.convert_element_type(
                    jnp.minimum(ivb, 1), jnp.float32
                )
                wb = wf * hif
                wa = wf - wb
                out = []
                for c in range(4):
                    va = rows_v[f - f_lo, pl.ds(c * 16, 16)]
                    vb = rows_v[f - f_lo, pl.ds(64 + c * 16, 16)]
                    out.append(acc[c] + va * wa + vb * wb)
                return tuple(out)

            if h0 > 0:
                init = tuple(acc_v[b, pl.ds(c * 16, 16)] for c in range(4))
            else:
                init = tuple(jnp.zeros((16,), jnp.float32) for _ in range(4))

            n = h1 - h0
            base = b * _HIST + h0
            nu, rem = divmod(n, 5)

            def _acc5(i, acc, base=base, _acc1=_acc1):
                f = base + i * 5
                for k in range(5):
                    acc = _acc1(f + k, acc)
                return acc

            acc = lax.fori_loop(0, nu, _acc5, init) if nu else init
            for k in range(n - rem, n):
                acc = _acc1(base + k, acc)
            for c in range(4):
                acc_v[b, pl.ds(c * 16, 16)] = acc[c]

    pltpu.sync_copy(acc_v, out_hbm.at[pl.ds(wid * _BPW, _BPW)])


def _pool(table_lines, idx3):
    mesh = plsc.VectorSubcoreMesh(core_axis_name="c", subcore_axis_name="s")
    return pl.kernel(
        _pool_body,
        mesh=mesh,
        out_type=jax.ShapeDtypeStruct((_BATCH, _EMB), jnp.float32),
        scratch_types=[
            pltpu.VMEM((_NCPAD, _CHUNK), jnp.int32),
            pltpu.VMEM((_NCPAD, _CHUNK), jnp.int32),
            pltpu.VMEM((_SBUF, 2 * _EMB), jnp.float32),
            pltpu.VMEM((_BPW, _EMB), jnp.float32),
            pltpu.SemaphoreType.DMA,
        ],
    )(table_lines, idx3)


_RB = 2048
_NR = 25
_NLINES = _NR * _RB


def _relayout_body(a_ref, b_ref, o_ref):
    o_ref[:, 0:_EMB] = jnp.transpose(a_ref[...])
    o_ref[:, _EMB : 2 * _EMB] = jnp.transpose(b_ref[...])


def _to_lines(emb_table_t):
    return pl.pallas_call(
        _relayout_body,
        grid=(_NR,),
        in_specs=[
            pl.BlockSpec((_EMB, _RB), lambda i: (0, 2 * i)),
            pl.BlockSpec((_EMB, _RB), lambda i: (0, jnp.minimum(2 * i + 1, 48))),
        ],
        out_specs=pl.BlockSpec((_RB, 2 * _EMB), lambda i: (i, 0)),
        out_shape=jax.ShapeDtypeStruct((_NLINES, 2 * _EMB), jnp.float32),
    )(emb_table_t, emb_table_t)


_VT = 4096
_NV = (_VOCAB + _VT - 1) // _VT


def _mm_body(x_ref, m_ref, w_ref, b_ref, o_ref):
    xd = x_ref[...] * m_ref[...]
    acc = lax.dot_general(
        w_ref[...], xd, (((0,), (1,)), ((), ())), preferred_element_type=jnp.float32
    )
    j = pl.program_id(0)
    bias = b_ref[0, pl.ds(j * _VT, _VT)]
    o_ref[...] = acc + bias[:, None]


def _project(pooled, mscale, fc_Wt, bias_padded):
    out_t = pl.pallas_call(
        _mm_body,
        grid=(_NV,),
        in_specs=[
            pl.BlockSpec((_BATCH, _EMB), lambda j: (0, 0)),
            pl.BlockSpec((_BATCH, _EMB), lambda j: (0, 0)),
            pl.BlockSpec((_EMB, _VT), lambda j: (0, j)),
            pl.BlockSpec((1, _NV * _VT), lambda j: (0, 0)),
        ],
        out_specs=pl.BlockSpec((_VT, _BATCH), lambda j: (j, 0)),
        out_shape=jax.ShapeDtypeStruct((_VOCAB, _BATCH), jnp.float32),
    )(pooled, mscale, fc_Wt, bias_padded)
    return out_t.T


def kernel(x_in, emb_table, fc_W, fc_b):
    x = x_in.astype(jnp.int32)
    npad = _NROWS - _IDX_PER_W
    pad = (
        jnp.arange(_NW, dtype=jnp.int32)[:, None] * npad
        + jnp.arange(npad, dtype=jnp.int32)[None, :]
    )
    idx3 = jnp.concatenate([x.reshape(_NW, _IDX_PER_W), pad], axis=1).reshape(
        _NW, _NCHUNK, _CHUNK
    )
    idx3 = jnp.concatenate(
        [idx3, jnp.zeros((_NW, _NCPAD - _NCHUNK, _CHUNK), jnp.int32)], axis=1
    )
    pooled = _pool(_to_lines(emb_table.T), idx3)

    keep = jax.random.bernoulli(jax.random.key(42), _KEEP_P, (_BATCH, _EMB))
    mscale = jnp.where(keep, jnp.float32(1.0 / _KEEP_P), jnp.float32(0.0))

    bias_padded = jnp.pad(fc_b, (0, _NV * _VT - _VOCAB)).reshape(1, -1)
    return _project(pooled, mscale, fc_W.T, bias_padded)
